# Initial kernel scaffold; baseline (speedup 1.0000x reference)
import jax
import jax.numpy as jnp
from jax.experimental import pallas as pl


def kernel(node_state, graph_indicator):
    raise NotImplementedError("write your pallas kernel here")


# speedup vs baseline: 2.8619x; 2.8619x over previous
import functools

import jax
import jax.numpy as jnp
from jax import lax
from jax.experimental import pallas as pl
from jax.experimental.pallas import tpu as pltpu
from jax.experimental.pallas import tpu_sc as plsc

NSEG = 512
D = 128
NLANE = 16
NF = D // NLANE
NW = 32
SEG_PER_W = NSEG // NW
BLK = 32


def _sc_body(nrows, ns_hbm, starts_hbm, out_hbm, starts_v, buf_v, outbuf_v, sems):
    cid = lax.axis_index("c")
    sid = lax.axis_index("s")
    wid = sid * 2 + cid
    seg0 = wid * SEG_PER_W

    pltpu.sync_copy(starts_hbm.at[pl.ds(seg0, 32)], starts_v)

    def seg_body(j, _):
        start = starts_v[pl.ds(j, NLANE)][0]
        end = starts_v[pl.ds(j + 1, NLANE)][0]
        cnt = end - start
        p0 = (start // 8) * 8
        nblk = (end - p0 + BLK - 1) // BLK

        def p_fetch(i):
            return jnp.minimum(p0 + i * BLK, nrows - BLK)

        def issue(i, slot):
            pltpu.make_async_copy(
                ns_hbm.at[pl.ds(p_fetch(i), BLK)], buf_v.at[slot], sems.at[slot]
            ).start()

        @pl.when(nblk > 0)
        def _prologue():
            issue(0, 0)

        def blk_body(i, acc):
            slot = lax.rem(i, 2)
            p_nom = p0 + i * BLK
            p_i = jnp.minimum(p_nom, nrows - BLK)
            pltpu.make_async_copy(
                ns_hbm.at[pl.ds(p_i, BLK)], buf_v.at[slot], sems.at[slot]
            ).wait()

            @pl.when(i + 1 < nblk)
            def _next():
                issue(i + 1, 1 - slot)

            rlo = jnp.maximum(start, p_nom) - p_i
            rhi = end - p_i
            accl = list(acc)
            for r in range(BLK):
                m = jnp.where(jnp.logical_and(r >= rlo, r < rhi), 1.0, 0.0)
                for f in range(NF):
                    v = buf_v[slot, r, pl.ds(f * NLANE, NLANE)]
                    accl[f] = accl[f] + v * m
            return tuple(accl)

        zero = jnp.zeros((NLANE,), jnp.float32)
        acc = lax.fori_loop(0, nblk, blk_body, (zero,) * NF)

        denom = jnp.maximum(cnt, 1).astype(jnp.float32)
        bits = lax.bitcast_convert_type(denom, jnp.int32)
        recip = lax.bitcast_convert_type(jnp.int32(0x7EF127EA) - bits, jnp.float32)
        for _ in range(3):
            recip = recip * (2.0 - denom * recip)
        for f in range(NF):
            outbuf_v[j, pl.ds(f * NLANE, NLANE)] = acc[f] * recip
        return 0

    lax.fori_loop(0, SEG_PER_W, seg_body, 0)
    pltpu.sync_copy(outbuf_v, out_hbm.at[pl.ds(seg0, SEG_PER_W)])


def _make_sc_kernel(nrows):
    mesh = plsc.VectorSubcoreMesh(core_axis_name="c", subcore_axis_name="s")
    return pl.kernel(
        functools.partial(_sc_body, nrows),
        out_type=jax.ShapeDtypeStruct((NSEG, D), jnp.float32),
        mesh=mesh,
        scratch_types=[
            pltpu.VMEM((32,), jnp.int32),
            pltpu.VMEM((2, BLK, D), jnp.float32),
            pltpu.VMEM((SEG_PER_W, D), jnp.float32),
            pltpu.SemaphoreType.DMA((2,)),
        ],
    )


@jax.jit
def kernel(node_state, graph_indicator):
    nrows = node_state.shape[0]
    seg = graph_indicator.astype(jnp.int32)
    starts = jnp.searchsorted(
        seg, jnp.arange(NSEG, dtype=jnp.int32), side="left"
    ).astype(jnp.int32)
    starts = jnp.concatenate(
        [starts, jnp.full((16,), nrows, dtype=jnp.int32)]
    )
    return _make_sc_kernel(nrows)(node_state, starts)


# speedup vs baseline: 4.3656x; 1.5254x over previous
import jax
import jax.numpy as jnp
from jax import lax
from jax.experimental import pallas as pl
from jax.experimental.pallas import tpu as pltpu
from jax.experimental.pallas import tpu_sc as plsc

NSEG = 512
D = 128
NLANE = 16
NF = D // NLANE
NW = 32
SEG_PER_W = NSEG // NW
NB = SEG_PER_W + 1
BLK = 32
NRING = 2
NROWS = 100000
IDR = 128
NID_ROWS = (NROWS + IDR - 1) // IDR
NCOARSE = 800
SEARCH_STEPS = 10


def _sc_body(ns_hbm, seg_hbm, coarse_hbm, out_hbm, starts_v, coarse_v,
             rows_v, bounce_v, outbuf_v, buf_v, gsem, sems):
    cid = lax.axis_index("c")
    sid = lax.axis_index("s")
    wid = sid * 2 + cid
    seg0 = wid * SEG_PER_W
    lanes = lax.iota(jnp.int32, NLANE)

    pltpu.sync_copy(coarse_hbm, coarse_v)

    def lane_sum(vec):
        bounce_v[pl.ds(0, NLANE)] = vec
        tot = jnp.int32(0)
        for e in range(NLANE):
            tot = tot + bounce_v[pl.ds(e, NLANE)][0]
        return tot

    cnt_vecs = [jnp.zeros((NLANE,), jnp.int32) for _ in range(NB)]
    for c in range(NCOARSE // NLANE):
        chunk = coarse_v[pl.ds(c * NLANE, NLANE)]
        for k in range(NB):
            cnt_vecs[k] = cnt_vecs[k] + jnp.where(chunk < seg0 + k, 1, 0)
    los = [lane_sum(cnt_vecs[k]) for k in range(NB)]

    def row_copy(k):
        src = seg_hbm.at[pl.ds(jnp.maximum(los[k] - 1, 0) * IDR, IDR)]
        return pltpu.make_async_copy(src, rows_v.at[k], gsem)

    for k in range(NB):
        row_copy(k).start()
    for k in range(NB):
        row_copy(k).wait()

    bounds = []
    for k in range(NB):
        t_k = seg0 + k
        cvec = jnp.zeros((NLANE,), jnp.int32)
        for f in range(NF):
            chunk = rows_v[k, pl.ds(f * NLANE, NLANE)]
            cvec = cvec + jnp.where(chunk < t_k, 1, 0)
        c_k = lane_sum(cvec)
        bounds.append(jnp.maximum((los[k] - 1) * IDR + c_k, 0))

    bvec = jnp.zeros((NLANE,), jnp.int32)
    for k in range(NLANE):
        bvec = jnp.where(lanes == k, bounds[k], bvec)
    starts_v[pl.ds(0, NLANE)] = bvec
    starts_v[pl.ds(NLANE, NLANE)] = jnp.where(lanes == 0, bounds[NLANE], 0)

    def seg_body(j, _):
        start = starts_v[pl.ds(j, NLANE)][0]
        end = starts_v[pl.ds(j + 1, NLANE)][0]
        cnt = end - start
        p0 = (start // 8) * 8
        nblk = (end - p0 + BLK - 1) // BLK

        def p_fetch(i):
            return jnp.minimum(p0 + i * BLK, NROWS - BLK)

        def issue(i, slot):
            pltpu.make_async_copy(
                ns_hbm.at[pl.ds(p_fetch(i), BLK)], buf_v.at[slot], sems.at[slot]
            ).start()

        for k in range(NRING):
            @pl.when(k < nblk)
            def _prologue():
                issue(k, k)

        def blk_body(i, acc):
            slot = lax.rem(i, NRING)
            p_nom = p0 + i * BLK
            p_i = jnp.minimum(p_nom, NROWS - BLK)
            pltpu.make_async_copy(
                ns_hbm.at[pl.ds(p_i, BLK)], buf_v.at[slot], sems.at[slot]
            ).wait()

            rlo = jnp.maximum(start, p_nom) - p_i
            rhi = end - p_i
            accl = list(acc)
            for r in range(BLK):
                m = jnp.where(jnp.logical_and(r >= rlo, r < rhi), 1.0, 0.0)
                for f in range(NF):
                    v = buf_v[slot, r, pl.ds(f * NLANE, NLANE)]
                    accl[f] = accl[f] + v * m

            @pl.when(i + NRING < nblk)
            def _refill():
                issue(i + NRING, slot)

            return tuple(accl)

        zero = jnp.zeros((NLANE,), jnp.float32)
        acc = lax.fori_loop(0, nblk, blk_body, (zero,) * NF)

        denom = jnp.maximum(cnt, 1).astype(jnp.float32)
        bits = lax.bitcast_convert_type(denom, jnp.int32)
        recip = lax.bitcast_convert_type(jnp.int32(0x7EF127EA) - bits, jnp.float32)
        for _ in range(3):
            recip = recip * (2.0 - denom * recip)

        for f in range(NF):
            outbuf_v[j, pl.ds(f * NLANE, NLANE)] = acc[f] * recip
        return 0

    lax.fori_loop(0, SEG_PER_W, seg_body, 0)
    pltpu.sync_copy(outbuf_v, out_hbm.at[pl.ds(seg0, SEG_PER_W)])


def _make_sc_kernel():
    mesh = plsc.VectorSubcoreMesh(core_axis_name="c", subcore_axis_name="s")
    return pl.kernel(
        _sc_body,
        out_type=jax.ShapeDtypeStruct((NSEG, D), jnp.float32),
        mesh=mesh,
        scratch_types=[
            pltpu.VMEM((2 * NLANE,), jnp.int32),
            pltpu.VMEM((NCOARSE,), jnp.int32),
            pltpu.VMEM((NB, IDR), jnp.int32),
            pltpu.VMEM((2 * NLANE,), jnp.int32),
            pltpu.VMEM((SEG_PER_W, D), jnp.float32),
            pltpu.VMEM((NRING, BLK, D), jnp.float32),
            pltpu.SemaphoreType.DMA,
            pltpu.SemaphoreType.DMA((NRING,)),
        ],
    )


@jax.jit
def kernel(node_state, graph_indicator):
    seg = graph_indicator.astype(jnp.int32)
    pad = jnp.full((NID_ROWS * IDR - NROWS,), NSEG, jnp.int32)
    seg1 = jnp.concatenate([seg, pad])
    coarse = seg1[:: IDR]
    coarse = jnp.concatenate(
        [coarse, jnp.full((NCOARSE - NID_ROWS,), NSEG, jnp.int32)]
    )
    return _make_sc_kernel()(node_state, seg1, coarse)
